# Initial kernel scaffold; baseline (speedup 1.0000x reference)
import functools

import jax
import jax.numpy as jnp
from jax import lax
from jax.experimental import pallas as pl
from jax.experimental.pallas import tpu as pltpu
from jax.experimental.pallas import tpu_sc as plsc

_NC = 2
_NS = 16
_NW = _NC * _NS
_C = 128


def _sc_agg_deg_body(x_hbm, srcr, dstr, z_row, z_deg, ones_h,
                     agg_out, deg_out,
                     idx_s, idx_d, rows, ones_v, acc_sh, deg_sh):
  K = idx_s.shape[0]
  cid = lax.axis_index("c")
  sid = lax.axis_index("s")
  wid = sid * _NC + cid
  nt = acc_sh.shape[0] // _NS
  pltpu.sync_copy(z_row, acc_sh.at[pl.ds(sid * nt, nt)])
  pltpu.sync_copy(z_deg, deg_sh.at[pl.ds(sid * nt, nt)])
  pltpu.sync_copy(ones_h, ones_v)
  pltpu.sync_copy(srcr.at[wid], idx_s)
  pltpu.sync_copy(dstr.at[wid], idx_d)
  plsc.subcore_barrier()

  def step(j, carry):
    pltpu.sync_copy(x_hbm.at[idx_s.at[j]], rows)
    pltpu.sync_copy(rows, acc_sh.at[idx_d.at[j]], add=True)
    pltpu.sync_copy(ones_v, deg_sh.at[idx_d.at[j]], add=True)
    return carry

  lax.fori_loop(0, K, step, 0)
  plsc.subcore_barrier()
  pltpu.sync_copy(acc_sh.at[pl.ds(sid * nt, nt)],
                  agg_out.at[cid, pl.ds(sid * nt, nt)])
  pltpu.sync_copy(deg_sh.at[pl.ds(sid * nt, nt)],
                  deg_out.at[cid, pl.ds(sid * nt, nt)])


def _sc_agg_body(x_hbm, srcr, dstr, z_row,
                 agg_out,
                 idx_s, idx_d, rows, acc_sh):
  K = idx_s.shape[0]
  cid = lax.axis_index("c")
  sid = lax.axis_index("s")
  wid = sid * _NC + cid
  nt = acc_sh.shape[0] // _NS
  pltpu.sync_copy(z_row, acc_sh.at[pl.ds(sid * nt, nt)])
  pltpu.sync_copy(srcr.at[wid], idx_s)
  pltpu.sync_copy(dstr.at[wid], idx_d)
  plsc.subcore_barrier()

  def step(j, carry):
    pltpu.sync_copy(x_hbm.at[idx_s.at[j]], rows)
    pltpu.sync_copy(rows, acc_sh.at[idx_d.at[j]], add=True)
    return carry

  lax.fori_loop(0, K, step, 0)
  plsc.subcore_barrier()
  pltpu.sync_copy(acc_sh.at[pl.ds(sid * nt, nt)],
                  agg_out.at[cid, pl.ds(sid * nt, nt)])


def _make_sc_agg_deg(nacc, K, D):
  mesh = plsc.VectorSubcoreMesh(core_axis_name="c", subcore_axis_name="s")
  return pl.kernel(
      _sc_agg_deg_body,
      mesh=mesh,
      out_type=[
          jax.ShapeDtypeStruct((_NC, nacc, D), jnp.float32),
          jax.ShapeDtypeStruct((_NC, nacc, 16), jnp.float32),
      ],
      scratch_types=[
          pltpu.VMEM((K, _C), jnp.int32),
          pltpu.VMEM((K, _C), jnp.int32),
          pltpu.VMEM((_C, D), jnp.float32),
          pltpu.VMEM((_C, 16), jnp.float32),
          pltpu.VMEM_SHARED((nacc, D), jnp.float32),
          pltpu.VMEM_SHARED((nacc, 16), jnp.float32),
      ],
  )


def _make_sc_agg(nacc, K, D):
  mesh = plsc.VectorSubcoreMesh(core_axis_name="c", subcore_axis_name="s")
  return pl.kernel(
      _sc_agg_body,
      mesh=mesh,
      out_type=jax.ShapeDtypeStruct((_NC, nacc, D), jnp.float32),
      scratch_types=[
          pltpu.VMEM((K, _C), jnp.int32),
          pltpu.VMEM((K, _C), jnp.int32),
          pltpu.VMEM((_C, D), jnp.float32),
          pltpu.VMEM_SHARED((nacc, D), jnp.float32),
      ],
  )


def _dotT(a, w):
  return lax.dot_general(a, w, (((1,), (1,)), ((), ())),
                         preferred_element_type=jnp.float32)


def _tc1_body(aggp, degp, x, w1l, b1l, w1r, h_out):
  agg = aggp[0] + aggp[1]
  deg = degp[0, :, 0:1] + degp[1, :, 0:1]
  a = agg / jnp.maximum(deg, 1.0)
  y = _dotT(a, w1l[...]) + b1l[...] + _dotT(x[...], w1r[...])
  h_out[...] = jnp.maximum(y, 0.0)


def _tc2_body(aggp, degp, h, img, attr, rel,
              w2l, b2l, w2r, wg, bg, wi, bi, wa, ba, wr, br, wf, bf,
              fused_out, h2_out, im_out, at_out, re_out):
  agg = aggp[0] + aggp[1]
  deg = degp[0, :, 0:1] + degp[1, :, 0:1]
  a = agg / jnp.maximum(deg, 1.0)
  h2 = _dotT(a, w2l[...]) + b2l[...] + _dotT(h[...], w2r[...])
  g = _dotT(h2, wg[...]) + bg[...]
  im = _dotT(img[...], wi[...]) + bi[...]
  at = _dotT(attr[...], wa[...]) + ba[...]
  re = _dotT(rel[...], wr[...]) + br[...]

  wfv = wf[...]
  b = bf[:, 0:1]
  lg = jnp.sum(g * wfv, axis=1, keepdims=True) + b
  li = jnp.sum(im * wfv, axis=1, keepdims=True) + b
  la = jnp.sum(at * wfv, axis=1, keepdims=True) + b
  lr = jnp.sum(re * wfv, axis=1, keepdims=True) + b
  m = jnp.maximum(jnp.maximum(lg, li), jnp.maximum(la, lr))
  eg = jnp.exp(lg - m)
  ei = jnp.exp(li - m)
  ea = jnp.exp(la - m)
  er = jnp.exp(lr - m)
  s = eg + ei + ea + er
  fused = (eg * g + ei * im + ea * at + er * re) / s
  nrm = jnp.sqrt(jnp.sum(fused * fused, axis=1, keepdims=True))
  fused_out[...] = fused / jnp.maximum(nrm, 1e-12)
  h2_out[...] = h2
  im_out[...] = im
  at_out[...] = at
  re_out[...] = re


def _row_spec(R, D):
  return pl.BlockSpec((R, D), lambda i: (i, 0))


def _full_spec(shape):
  nd = len(shape)
  return pl.BlockSpec(shape, lambda i: (0,) * nd)


def _tc1(aggp, degp, x, w1l, b1l, w1r):
  N, D = x.shape
  R = 1000
  return pl.pallas_call(
      _tc1_body,
      grid=(N // R,),
      in_specs=[
          pl.BlockSpec((_NC, R, D), lambda i: (0, i, 0)),
          pl.BlockSpec((_NC, R, 16), lambda i: (0, i, 0)),
          _row_spec(R, D),
          _full_spec((D, D)),
          _full_spec((1, D)),
          _full_spec((D, D)),
      ],
      out_specs=_row_spec(R, D),
      out_shape=jax.ShapeDtypeStruct((N, D), jnp.float32),
  )(aggp, degp, x, w1l, b1l, w1r)


def _tc2(aggp, degp, h, img, attr, rel, *ws):
  N, D = h.shape
  R = 1000
  out = jax.ShapeDtypeStruct((N, D), jnp.float32)
  w_specs = [
      _full_spec((D, D)), _full_spec((1, D)), _full_spec((D, D)),
      _full_spec((D, D)), _full_spec((1, D)),
      _full_spec((D, D)), _full_spec((1, D)),
      _full_spec((D, D)), _full_spec((1, D)),
      _full_spec((D, D)), _full_spec((1, D)),
      _full_spec((1, D)), _full_spec((1, D)),
  ]
  return pl.pallas_call(
      _tc2_body,
      grid=(N // R,),
      in_specs=[
          pl.BlockSpec((_NC, R, D), lambda i: (0, i, 0)),
          pl.BlockSpec((_NC, R, 16), lambda i: (0, i, 0)),
          _row_spec(R, D), _row_spec(R, D), _row_spec(R, D), _row_spec(R, D),
      ] + w_specs,
      out_specs=[_row_spec(R, D)] * 5,
      out_shape=[out, out, out, out, out],
  )(aggp, degp, h, img, attr, rel, *ws)


def kernel(x, edge_index, img_emb, attr_emb, rel_emb,
           W1l, b1l, W1r, W2l, b2l, W2r, Wgph, bgph, Wimg, bimg,
           Watt, batt, Wrel, brel, Wfus, bfus):
  N, D = x.shape
  E = edge_index.shape[1]

  K = -(-E // (_NW * _C))
  pad = _NW * K * _C - E
  src = edge_index[0].astype(jnp.int32)
  dst = edge_index[1].astype(jnp.int32)
  if pad:
    ar = jnp.arange(pad, dtype=jnp.int32)
    src = jnp.concatenate([src, (ar * 97) % N])
    dst = jnp.concatenate([dst, N + (ar % 32)])
  srcr = src.reshape(_NW, K, _C)
  dstr = dst.reshape(_NW, K, _C)

  nacc = ((N + 32 + _NS - 1) // _NS) * _NS
  nt = nacc // _NS
  z_row = jnp.zeros((nt, D), jnp.float32)
  z_deg = jnp.zeros((nt, 16), jnp.float32)
  ones_h = jnp.ones((_C, 16), jnp.float32)

  agg1p, degp = _make_sc_agg_deg(nacc, K, D)(
      x, srcr, dstr, z_row, z_deg, ones_h)
  h = _tc1(agg1p, degp, x, W1l, b1l.reshape(1, D), W1r)
  agg2p = _make_sc_agg(nacc, K, D)(h, srcr, dstr, z_row)
  fused, h2, im, at, re = _tc2(
      agg2p, degp, h, img_emb, attr_emb, rel_emb,
      W2l, b2l.reshape(1, D), W2r,
      Wgph, bgph.reshape(1, D),
      Wimg, bimg.reshape(1, D),
      Watt, batt.reshape(1, D),
      Wrel, brel.reshape(1, D),
      Wfus, jnp.broadcast_to(bfus.reshape(1, 1), (1, D)))
  return fused, h2, im, at, re


# speedup vs baseline: 8.1317x; 8.1317x over previous
import jax
import jax.numpy as jnp
from jax import lax
from jax.experimental import pallas as pl
from jax.experimental.pallas import tpu as pltpu
from jax.experimental.pallas import tpu_sc as plsc

_NC = 2
_NS = 16
_NW = _NC * _NS
_C = 128
_KB = 16


def _sc_agg_body(x_hbm, srcr, dstr, z_row, agg_out,
                 idx_s, idx_d, rows0, rows1, acc_sh, sem0, sem1):
  ko_n = srcr.shape[1]
  cid = lax.axis_index("c")
  sid = lax.axis_index("s")
  wid = sid * _NC + cid
  nt = acc_sh.shape[0] // _NS
  pltpu.sync_copy(z_row, acc_sh.at[pl.ds(sid * nt, nt)])
  plsc.subcore_barrier()

  def outer(ko, carry):
    pltpu.sync_copy(srcr.at[wid, ko], idx_s)
    pltpu.sync_copy(dstr.at[wid, ko], idx_d)
    pltpu.async_copy(x_hbm.at[idx_s.at[0]], rows0, sem0)

    def inner(j2, c2):
      e = 2 * j2
      pltpu.async_copy(x_hbm.at[idx_s.at[e + 1]], rows1, sem1)
      pltpu.make_async_copy(x_hbm.at[idx_s.at[e]], rows0, sem0).wait()
      pltpu.sync_copy(rows0, acc_sh.at[idx_d.at[e]], add=True)

      @pl.when(j2 + 1 < _KB // 2)
      def _():
        pltpu.async_copy(x_hbm.at[idx_s.at[e + 2]], rows0, sem0)

      pltpu.make_async_copy(x_hbm.at[idx_s.at[e + 1]], rows1, sem1).wait()
      pltpu.sync_copy(rows1, acc_sh.at[idx_d.at[e + 1]], add=True)
      return c2

    return lax.fori_loop(0, _KB // 2, inner, carry)

  lax.fori_loop(0, ko_n, outer, 0)
  plsc.subcore_barrier()
  pltpu.sync_copy(acc_sh.at[pl.ds(sid * nt, nt)],
                  agg_out.at[cid, pl.ds(sid * nt, nt)])


def _make_sc_agg(nacc, D):
  mesh = plsc.VectorSubcoreMesh(core_axis_name="c", subcore_axis_name="s")
  return pl.kernel(
      _sc_agg_body,
      mesh=mesh,
      out_type=jax.ShapeDtypeStruct((_NC, nacc, D), jnp.float32),
      scratch_types=[
          pltpu.VMEM((_KB, _C), jnp.int32),
          pltpu.VMEM((_KB, _C), jnp.int32),
          pltpu.VMEM((_C, D), jnp.float32),
          pltpu.VMEM((_C, D), jnp.float32),
          pltpu.VMEM_SHARED((nacc, D), jnp.float32),
          pltpu.SemaphoreType.DMA,
          pltpu.SemaphoreType.DMA,
      ],
  )


def _dotT(a, w):
  return lax.dot_general(a, w, (((1,), (1,)), ((), ())),
                         preferred_element_type=jnp.float32)


def _tc1_body(aggp, degp, x, w1l, b1l, w1r, h_out):
  agg = aggp[0] + aggp[1]
  deg = degp[0, :, 0:1] + degp[1, :, 0:1]
  a = agg / jnp.maximum(deg, 1.0)
  y = _dotT(a, w1l[...]) + b1l[...] + _dotT(x[...], w1r[...])
  h_out[...] = jnp.maximum(y, 0.0)


def _tc2_body(aggp, degp, h, img, attr, rel,
              w2l, b2l, w2r, wg, bg, wi, bi, wa, ba, wr, br, wf, bf,
              fused_out, h2_out, im_out, at_out, re_out):
  agg = aggp[0] + aggp[1]
  deg = degp[0, :, 0:1] + degp[1, :, 0:1]
  a = agg / jnp.maximum(deg, 1.0)
  h2 = _dotT(a, w2l[...]) + b2l[...] + _dotT(h[...], w2r[...])
  g = _dotT(h2, wg[...]) + bg[...]
  im = _dotT(img[...], wi[...]) + bi[...]
  at = _dotT(attr[...], wa[...]) + ba[...]
  re = _dotT(rel[...], wr[...]) + br[...]

  wfv = wf[...]
  b = bf[:, 0:1]
  lg = jnp.sum(g * wfv, axis=1, keepdims=True) + b
  li = jnp.sum(im * wfv, axis=1, keepdims=True) + b
  la = jnp.sum(at * wfv, axis=1, keepdims=True) + b
  lr = jnp.sum(re * wfv, axis=1, keepdims=True) + b
  m = jnp.maximum(jnp.maximum(lg, li), jnp.maximum(la, lr))
  eg = jnp.exp(lg - m)
  ei = jnp.exp(li - m)
  ea = jnp.exp(la - m)
  er = jnp.exp(lr - m)
  s = eg + ei + ea + er
  fused = (eg * g + ei * im + ea * at + er * re) / s
  nrm = jnp.sqrt(jnp.sum(fused * fused, axis=1, keepdims=True))
  fused_out[...] = fused / jnp.maximum(nrm, 1e-12)
  h2_out[...] = h2
  im_out[...] = im
  at_out[...] = at
  re_out[...] = re


def _row_spec(R, D):
  return pl.BlockSpec((R, D), lambda i: (i, 0))


def _full_spec(shape):
  nd = len(shape)
  return pl.BlockSpec(shape, lambda i: (0,) * nd)


def _tc1(aggp, degp, x, w1l, b1l, w1r):
  N, D = x.shape
  R = 1000
  return pl.pallas_call(
      _tc1_body,
      grid=(N // R,),
      in_specs=[
          pl.BlockSpec((_NC, R, D), lambda i: (0, i, 0)),
          pl.BlockSpec((_NC, R, D), lambda i: (0, i, 0)),
          _row_spec(R, D),
          _full_spec((D, D)),
          _full_spec((1, D)),
          _full_spec((D, D)),
      ],
      out_specs=_row_spec(R, D),
      out_shape=jax.ShapeDtypeStruct((N, D), jnp.float32),
  )(aggp, degp, x, w1l, b1l, w1r)


def _tc2(aggp, degp, h, img, attr, rel, *ws):
  N, D = h.shape
  R = 1000
  out = jax.ShapeDtypeStruct((N, D), jnp.float32)
  w_specs = [
      _full_spec((D, D)), _full_spec((1, D)), _full_spec((D, D)),
      _full_spec((D, D)), _full_spec((1, D)),
      _full_spec((D, D)), _full_spec((1, D)),
      _full_spec((D, D)), _full_spec((1, D)),
      _full_spec((D, D)), _full_spec((1, D)),
      _full_spec((1, D)), _full_spec((1, D)),
  ]
  return pl.pallas_call(
      _tc2_body,
      grid=(N // R,),
      in_specs=[
          pl.BlockSpec((_NC, R, D), lambda i: (0, i, 0)),
          pl.BlockSpec((_NC, R, D), lambda i: (0, i, 0)),
          _row_spec(R, D), _row_spec(R, D), _row_spec(R, D), _row_spec(R, D),
      ] + w_specs,
      out_specs=[_row_spec(R, D)] * 5,
      out_shape=[out, out, out, out, out],
  )(aggp, degp, h, img, attr, rel, *ws)


def kernel(x, edge_index, img_emb, attr_emb, rel_emb,
           W1l, b1l, W1r, W2l, b2l, W2r, Wgph, bgph, Wimg, bimg,
           Watt, batt, Wrel, brel, Wfus, bfus):
  N, D = x.shape
  E = edge_index.shape[1]

  ko_n = -(-E // (_NW * _KB * _C))
  pad = _NW * ko_n * _KB * _C - E
  src = edge_index[0].astype(jnp.int32)
  dst = edge_index[1].astype(jnp.int32)
  nacc = ((N + 96 + 8 * _NS - 1) // (8 * _NS)) * (8 * _NS)
  if pad:
    ar = jnp.arange(pad, dtype=jnp.int32)
    src = jnp.concatenate([src, (ar * 97) % N])
    dst = jnp.concatenate([dst, N + (ar % 96)])
  srcr = src.reshape(_NW, ko_n, _KB, _C)
  dstr = dst.reshape(_NW, ko_n, _KB, _C)

  nt = nacc // _NS
  z_row = jnp.zeros((nt, D), jnp.float32)

  sc_agg = _make_sc_agg(nacc, D)
  n_ones = 2048
  ones_tab = jnp.ones((n_ones, D), jnp.float32)
  arr = jnp.arange(_NW * ko_n * _KB * _C, dtype=jnp.int32)
  onesr = ((arr * 811) % n_ones).reshape(_NW, ko_n, _KB, _C)
  degp = sc_agg(ones_tab, onesr, dstr, z_row)
  agg1p = sc_agg(x, srcr, dstr, z_row)
  h = _tc1(agg1p, degp, x, W1l, b1l.reshape(1, D), W1r)
  agg2p = sc_agg(h, srcr, dstr, z_row)
  fused, h2, im, at, re = _tc2(
      agg2p, degp, h, img_emb, attr_emb, rel_emb,
      W2l, b2l.reshape(1, D), W2r,
      Wgph, bgph.reshape(1, D),
      Wimg, bimg.reshape(1, D),
      Watt, batt.reshape(1, D),
      Wrel, brel.reshape(1, D),
      Wfus, jnp.broadcast_to(bfus.reshape(1, 1), (1, D)))
  return fused, h2, im, at, re
